# baseline (speedup 1.0000x reference)
import functools

import jax
import jax.numpy as jnp
import numpy as np
from jax import lax
from jax.experimental import pallas as pl
from jax.experimental.pallas import tpu as pltpu
from jax.experimental.pallas import tpu_sc as plsc

FEAT = 128
HEADS = 2
N_RBF = 20
CUTOFF = 5.0
RBF_PAD = 32

_NC = 2
_NS = 16
_NW = _NC * _NS
_CHUNK = 128
_NBUF = 2


_LOG2E = 1.4426950408889634


def _swish(x):
    return x / (1.0 + jnp.exp2(x * -_LOG2E))


def _bf16_bits(x):
    b = lax.bitcast_convert_type(x, jnp.int32)
    return lax.shift_right_logical(
        b + 0x7FFF + (lax.shift_right_logical(b, 16) & 1), 16)


def _pack_pair(lo, hi):
    return _bf16_bits(lo) | lax.shift_left(_bf16_bits(hi), 16)


def _unpack_lo(w):
    return lax.bitcast_convert_type(lax.shift_left(w, 16), jnp.float32)


def _unpack_hi(w):
    return lax.bitcast_convert_type(w & jnp.int32(-65536), jnp.float32)


def _proj_body(x_ref, wq_ref, bq_ref, wkv_ref, bkv_ref, q_ref, kv_ref):
    x = x_ref[...]
    q = jnp.dot(x, wq_ref[...], preferred_element_type=jnp.float32) + bq_ref[...]
    kv = jnp.dot(x, wkv_ref[...], preferred_element_type=jnp.float32) + bkv_ref[...]
    q_ref[...] = _pack_pair(q[:, :FEAT], q[:, FEAT:])
    kv_ref[:, :FEAT] = _pack_pair(kv[:, :FEAT], kv[:, FEAT:2 * FEAT])
    kv_ref[:, FEAT:] = _pack_pair(kv[:, 2 * FEAT:3 * FEAT], kv[:, 3 * FEAT:])


def _node_proj(x_i, wq_p, bq_p, wkv_p, bkv_p):
    n = x_i.shape[0]
    blk = 1000
    grid = n // blk
    return pl.pallas_call(
        _proj_body,
        grid=(grid,),
        in_specs=[
            pl.BlockSpec((blk, FEAT), lambda i: (i, 0)),
            pl.BlockSpec((FEAT, HEADS * FEAT), lambda i: (0, 0)),
            pl.BlockSpec((1, HEADS * FEAT), lambda i: (0, 0)),
            pl.BlockSpec((FEAT, 2 * HEADS * FEAT), lambda i: (0, 0)),
            pl.BlockSpec((1, 2 * HEADS * FEAT), lambda i: (0, 0)),
        ],
        out_specs=[
            pl.BlockSpec((blk, FEAT), lambda i: (i, 0)),
            pl.BlockSpec((blk, 2 * FEAT), lambda i: (i, 0)),
        ],
        out_shape=[
            jax.ShapeDtypeStruct((n, FEAT), jnp.int32),
            jax.ShapeDtypeStruct((n, 2 * FEAT), jnp.int32),
        ],
    )(x_i, wq_p, bq_p, wkv_p, bkv_p)


def _make_gather(e_pad):
    epw = e_pad // _NW
    nch = epw // _CHUNK
    nouter = nch // _NBUF
    mesh = plsc.VectorSubcoreMesh(
        core_axis_name="c", subcore_axis_name="s",
        num_cores=_NC, num_subcores=_NS)

    dq = HEADS * FEAT // 2
    dkv = HEADS * FEAT

    @functools.partial(
        pl.kernel,
        mesh=mesh,
        out_type=[
            jax.ShapeDtypeStruct((e_pad, dq), jnp.int32),
            jax.ShapeDtypeStruct((e_pad, dkv), jnp.int32),
        ],
        scratch_types=[
            pltpu.VMEM((epw,), jnp.int32),
            pltpu.VMEM((epw,), jnp.int32),
            pltpu.VMEM((_NBUF, _CHUNK, dq), jnp.int32),
            pltpu.VMEM((_NBUF, _CHUNK, dkv), jnp.int32),
        ] + [pltpu.SemaphoreType.DMA] * (4 * _NBUF),
    )
    def gather_k(qtab, kvtab, idx_i, idx_j, qout, kvout,
                 idxi_v, idxj_v, qbuf, kvbuf, *sems):
        sq = sems[0:_NBUF]
        skv = sems[_NBUF:2 * _NBUF]
        wq = sems[2 * _NBUF:3 * _NBUF]
        wkv = sems[3 * _NBUF:4 * _NBUF]
        wid = lax.axis_index("s") * _NC + lax.axis_index("c")
        base = pl.multiple_of(wid * epw, _CHUNK)
        pltpu.sync_copy(idx_i.at[pl.ds(base, epw)], idxi_v)
        pltpu.sync_copy(idx_j.at[pl.ds(base, epw)], idxj_v)

        def issue(b, c):
            pltpu.async_copy(
                qtab.at[idxi_v.at[pl.ds(c * _CHUNK, _CHUNK)]], qbuf.at[b], sq[b])
            pltpu.async_copy(
                kvtab.at[idxj_v.at[pl.ds(c * _CHUNK, _CHUNK)]], kvbuf.at[b], skv[b])

        def wait_gather(b):
            pltpu.make_async_copy(
                qtab.at[idxi_v.at[pl.ds(0, _CHUNK)]], qbuf.at[b], sq[b]).wait()
            pltpu.make_async_copy(
                kvtab.at[idxj_v.at[pl.ds(0, _CHUNK)]], kvbuf.at[b], skv[b]).wait()

        def write(b, c):
            off = pl.multiple_of(base + c * _CHUNK, _CHUNK)
            pltpu.async_copy(qbuf.at[b], qout.at[pl.ds(off, _CHUNK)], wq[b])
            pltpu.async_copy(kvbuf.at[b], kvout.at[pl.ds(off, _CHUNK)], wkv[b])

        def wait_write(b):
            pltpu.make_async_copy(
                qbuf.at[b], qout.at[pl.ds(0, _CHUNK)], wq[b]).wait()
            pltpu.make_async_copy(
                kvbuf.at[b], kvout.at[pl.ds(0, _CHUNK)], wkv[b]).wait()

        for b in range(_NBUF):
            issue(b, b)

        def outer(g, carry):
            for b in range(_NBUF):
                wait_gather(b)
                write(b, g * _NBUF + b)
            for b in range(_NBUF):
                wait_write(b)
                nxt = (g + 1) * _NBUF + b

                @pl.when(nxt < nch)
                def _():
                    issue(b, nxt)
            return carry

        lax.fori_loop(0, nouter, outer, 0)

    return gather_k


_EPR = 4
_BR = 512


def _edge_body(step, inv2s2, *refs):
    if len(refs) == 7:
        refs = refs[1:]
    d_ref, qg_ref, kvg_ref, wd_ref, bd_ref, out_ref = refs
    f32, i32 = jnp.float32, jnp.int32
    be = out_ref.shape[0]
    br = be // _EPR
    mu = (lax.broadcasted_iota(i32, (1, _EPR * RBF_PAD), 1) & (RBF_PAD - 1)
          ).astype(f32) * step
    d4 = d_ref[...]
    bsel = (lax.broadcasted_iota(i32, (_EPR, _EPR * RBF_PAD), 1) // RBF_PAD
            == lax.broadcasted_iota(i32, (_EPR, _EPR * RBF_PAD), 0)).astype(f32)
    dm = jnp.dot(d4, bsel, preferred_element_type=f32,
                 precision=lax.Precision.HIGHEST)
    env = jnp.where(dm <= CUTOFF,
                    0.5 * (jnp.cos(np.pi * dm / CUTOFF) + 1.0), 0.0)
    ef4 = jnp.exp(-((dm - mu) ** 2) * inv2s2) * env
    dd4 = jnp.dot(ef4, wd_ref[...], preferred_element_type=f32) + bd_ref[...]
    dd4 = _swish(dd4)
    dd = jnp.concatenate(
        [jnp.concatenate(
            [dd4[:, (g * _EPR + s) * FEAT:(g * _EPR + s + 1) * FEAT]
             for g in range(4)], axis=1)
         for s in range(_EPR)], axis=0)
    qw = qg_ref[...]
    kvw = kvg_ref[...]
    kw = kvw[:, :FEAT]
    vw = kvw[:, FEAT:]
    prod0 = _unpack_lo(qw) * _unpack_lo(kw) * dd[:, :FEAT]
    prod1 = _unpack_hi(qw) * _unpack_hi(kw) * dd[:, FEAT:2 * FEAT]
    w0 = _swish(jnp.sum(prod0, axis=1, keepdims=True))
    w1 = _swish(jnp.sum(prod1, axis=1, keepdims=True))
    out_ref[:, :FEAT] = _unpack_lo(vw) * dd[:, 2 * FEAT:3 * FEAT] * w0
    out_ref[:, FEAT:] = _unpack_hi(vw) * dd[:, 3 * FEAT:] * w1


def _edge_stage(out_prev, blk0, e_out, dist4, qg, kvg, wd_p, bd_p):
    esl = qg.shape[0]
    be = _EPR * _BR
    nblk_total = (e_out + be - 1) // be
    grid = min(esl // be, nblk_total - blk0)
    if grid <= 0:
        return out_prev
    sigma = CUTOFF / (N_RBF - 1)
    body = functools.partial(_edge_body, float(sigma),
                             float(1.0 / (2.0 * sigma * sigma)))
    in_specs = [
        pl.BlockSpec((_BR, _EPR), lambda i: (i + blk0, 0)),
        pl.BlockSpec((be, FEAT), lambda i: (i, 0)),
        pl.BlockSpec((be, 2 * FEAT), lambda i: (i, 0)),
        pl.BlockSpec((_EPR * RBF_PAD, 4 * _EPR * FEAT), lambda i: (0, 0)),
        pl.BlockSpec((1, 4 * _EPR * FEAT), lambda i: (0, 0)),
    ]
    args = (dist4, qg, kvg, wd_p, bd_p)
    aliases = {}
    if out_prev is not None:
        in_specs = [pl.BlockSpec(memory_space=pl.ANY)] + in_specs
        args = (out_prev,) + args
        aliases = {0: 0}
    return pl.pallas_call(
        body,
        grid=(grid,),
        in_specs=in_specs,
        out_specs=pl.BlockSpec((be, HEADS * FEAT), lambda i: (i + blk0, 0)),
        out_shape=jax.ShapeDtypeStruct((e_out, HEADS * FEAT), jnp.float32),
        input_output_aliases=aliases,
    )(*args)


def _pack_node_w(w):
    return jnp.transpose(w, (2, 0, 1)).reshape(w.shape[2], w.shape[0] * w.shape[1])


def kernel(dist, nbrs, x_i, Wq, bq, Wk, bk, Wdk, bdk, Wv, bv, Wdv, bdv):
    e = dist.shape[0]
    n = x_i.shape[0]
    unit = _NW * _CHUNK * _NBUF
    e_pad = ((e + unit - 1) // unit) * unit
    sizes = []
    rem, cur = e_pad, unit
    while rem > 0:
        sz = min(cur, rem)
        sizes.append(sz)
        rem -= sz
        cur *= 2

    wq_p = _pack_node_w(Wq)
    wkv_p = jnp.concatenate([_pack_node_w(Wk), _pack_node_w(Wv)], axis=1)
    bq_p = bq.reshape(1, HEADS * FEAT)
    bkv_p = jnp.concatenate([bk.reshape(1, -1), bv.reshape(1, -1)], axis=1)

    wdk_p = _pack_node_w(Wdk)
    wdv_p = _pack_node_w(Wdv)
    wd_p = jnp.zeros((RBF_PAD, 2 * HEADS * FEAT), jnp.float32)
    wd_p = wd_p.at[:N_RBF].set(jnp.concatenate([wdk_p, wdv_p], axis=1))
    bd_p = jnp.concatenate([bdk.reshape(1, -1), bdv.reshape(1, -1)], axis=1)
    eye4 = jnp.eye(_EPR, dtype=jnp.float32)
    wd_r = wd_p.reshape(RBF_PAD, 4, FEAT)
    wblk = (eye4[:, None, None, :, None] * wd_r[None, :, :, None, :]).reshape(
        _EPR * RBF_PAD, 4 * _EPR * FEAT)
    bd4 = jnp.broadcast_to(bd_p.reshape(4, 1, FEAT), (4, _EPR, FEAT)).reshape(
        1, 4 * _EPR * FEAT)

    qtab_w, kvtab_w = _node_proj(x_i, wq_p, bq_p, wkv_p, bkv_p)

    idx = nbrs.astype(jnp.int32)
    pad = e_pad - e
    padvals = (jnp.arange(pad, dtype=jnp.int32) * 37) % n
    idx_i = jnp.concatenate([idx[:, 0], padvals])
    idx_j = jnp.concatenate([idx[:, 1], padvals])
    dist4 = jnp.pad(dist, (0, pad), constant_values=1.0).reshape(
        e_pad // (_EPR * _BR), _EPR, _BR).transpose(0, 2, 1).reshape(
        e_pad // _EPR, _EPR)

    gath_by_size = {}
    gathered = []
    lo = 0
    for sz in sizes:
        if sz not in gath_by_size:
            gath_by_size[sz] = _make_gather(sz)
        gathered.append(gath_by_size[sz](
            qtab_w, kvtab_w,
            lax.slice(idx_i, (lo,), (lo + sz,)),
            lax.slice(idx_j, (lo,), (lo + sz,))))
        lo += sz
    out = None
    blk0 = 0
    be = _EPR * _BR
    for sz, (qg_w, kvg_w) in zip(sizes, gathered):
        out = _edge_stage(out, blk0, e, dist4, qg_w, kvg_w, wblk, bd4)
        blk0 += sz // be
    return out


# speedup vs baseline: 1.0376x; 1.0376x over previous
import functools

import jax
import jax.numpy as jnp
import numpy as np
from jax import lax
from jax.experimental import pallas as pl
from jax.experimental.pallas import tpu as pltpu
from jax.experimental.pallas import tpu_sc as plsc

FEAT = 128
HEADS = 2
N_RBF = 20
CUTOFF = 5.0
RBF_PAD = 32

_NC = 2
_NS = 16
_NW = _NC * _NS
_CHUNK = 128
_NBUF = 2


_LOG2E = 1.4426950408889634


def _swish(x):
    return x / (1.0 + jnp.exp2(x * -_LOG2E))


def _bf16_bits(x):
    b = lax.bitcast_convert_type(x, jnp.int32)
    return lax.shift_right_logical(
        b + 0x7FFF + (lax.shift_right_logical(b, 16) & 1), 16)


def _pack_pair(lo, hi):
    return _bf16_bits(lo) | lax.shift_left(_bf16_bits(hi), 16)


def _unpack_lo(w):
    return lax.bitcast_convert_type(lax.shift_left(w, 16), jnp.float32)


def _unpack_hi(w):
    return lax.bitcast_convert_type(w & jnp.int32(-65536), jnp.float32)


def _proj_body(x_ref, wq_ref, bq_ref, wkv_ref, bkv_ref, q_ref, kv_ref):
    x = x_ref[...]
    q = jnp.dot(x, wq_ref[...], preferred_element_type=jnp.float32) + bq_ref[...]
    kv = jnp.dot(x, wkv_ref[...], preferred_element_type=jnp.float32) + bkv_ref[...]
    q_ref[...] = _pack_pair(q[:, :FEAT], q[:, FEAT:])
    kv_ref[:, :FEAT] = _pack_pair(kv[:, :FEAT], kv[:, FEAT:2 * FEAT])
    kv_ref[:, FEAT:] = _pack_pair(kv[:, 2 * FEAT:3 * FEAT], kv[:, 3 * FEAT:])


def _node_proj(x_i, wq_p, bq_p, wkv_p, bkv_p):
    n = x_i.shape[0]
    blk = 1000
    grid = n // blk
    return pl.pallas_call(
        _proj_body,
        grid=(grid,),
        in_specs=[
            pl.BlockSpec((blk, FEAT), lambda i: (i, 0)),
            pl.BlockSpec((FEAT, HEADS * FEAT), lambda i: (0, 0)),
            pl.BlockSpec((1, HEADS * FEAT), lambda i: (0, 0)),
            pl.BlockSpec((FEAT, 2 * HEADS * FEAT), lambda i: (0, 0)),
            pl.BlockSpec((1, 2 * HEADS * FEAT), lambda i: (0, 0)),
        ],
        out_specs=[
            pl.BlockSpec((blk, FEAT), lambda i: (i, 0)),
            pl.BlockSpec((blk, 2 * FEAT), lambda i: (i, 0)),
        ],
        out_shape=[
            jax.ShapeDtypeStruct((n, FEAT), jnp.int32),
            jax.ShapeDtypeStruct((n, 2 * FEAT), jnp.int32),
        ],
    )(x_i, wq_p, bq_p, wkv_p, bkv_p)


def _make_gather(e_pad):
    epw = e_pad // _NW
    nch = epw // _CHUNK
    nouter = nch // _NBUF
    mesh = plsc.VectorSubcoreMesh(
        core_axis_name="c", subcore_axis_name="s",
        num_cores=_NC, num_subcores=_NS)

    dq = HEADS * FEAT // 2
    dkv = HEADS * FEAT

    @functools.partial(
        pl.kernel,
        mesh=mesh,
        out_type=[
            jax.ShapeDtypeStruct((e_pad, dq), jnp.int32),
            jax.ShapeDtypeStruct((e_pad, dkv), jnp.int32),
        ],
        scratch_types=[
            pltpu.VMEM((epw,), jnp.int32),
            pltpu.VMEM((epw,), jnp.int32),
            pltpu.VMEM((_NBUF, _CHUNK, dq), jnp.int32),
            pltpu.VMEM((_NBUF, _CHUNK, dkv), jnp.int32),
        ] + [pltpu.SemaphoreType.DMA] * (4 * _NBUF),
    )
    def gather_k(qtab, kvtab, idx_i, idx_j, qout, kvout,
                 idxi_v, idxj_v, qbuf, kvbuf, *sems):
        sq = sems[0:_NBUF]
        skv = sems[_NBUF:2 * _NBUF]
        wq = sems[2 * _NBUF:3 * _NBUF]
        wkv = sems[3 * _NBUF:4 * _NBUF]
        wid = lax.axis_index("s") * _NC + lax.axis_index("c")
        base = pl.multiple_of(wid * epw, _CHUNK)
        pltpu.sync_copy(idx_i.at[pl.ds(base, epw)], idxi_v)
        pltpu.sync_copy(idx_j.at[pl.ds(base, epw)], idxj_v)

        def issue(b, c):
            pltpu.async_copy(
                qtab.at[idxi_v.at[pl.ds(c * _CHUNK, _CHUNK)]], qbuf.at[b], sq[b])
            pltpu.async_copy(
                kvtab.at[idxj_v.at[pl.ds(c * _CHUNK, _CHUNK)]], kvbuf.at[b], skv[b])

        def wait_gather(b):
            pltpu.make_async_copy(
                qtab.at[idxi_v.at[pl.ds(0, _CHUNK)]], qbuf.at[b], sq[b]).wait()
            pltpu.make_async_copy(
                kvtab.at[idxj_v.at[pl.ds(0, _CHUNK)]], kvbuf.at[b], skv[b]).wait()

        def write(b, c):
            off = pl.multiple_of(base + c * _CHUNK, _CHUNK)
            pltpu.async_copy(qbuf.at[b], qout.at[pl.ds(off, _CHUNK)], wq[b])
            pltpu.async_copy(kvbuf.at[b], kvout.at[pl.ds(off, _CHUNK)], wkv[b])

        def wait_write(b):
            pltpu.make_async_copy(
                qbuf.at[b], qout.at[pl.ds(0, _CHUNK)], wq[b]).wait()
            pltpu.make_async_copy(
                kvbuf.at[b], kvout.at[pl.ds(0, _CHUNK)], wkv[b]).wait()

        for b in range(_NBUF):
            issue(b, b)

        def outer(g, carry):
            for b in range(_NBUF):
                wait_gather(b)
                write(b, g * _NBUF + b)
            for b in range(_NBUF):
                wait_write(b)
                nxt = (g + 1) * _NBUF + b

                @pl.when(nxt < nch)
                def _():
                    issue(b, nxt)
            return carry

        lax.fori_loop(0, nouter, outer, 0)

    return gather_k


_EPR = 4
_BR = 512


def _edge_body(step, inv2s2, *refs):
    if len(refs) == 7:
        refs = refs[1:]
    d_ref, qg_ref, kvg_ref, wd_ref, bd_ref, out_ref = refs
    f32, i32 = jnp.float32, jnp.int32
    be = out_ref.shape[0]
    br = be // _EPR
    mu = (lax.broadcasted_iota(i32, (1, _EPR * RBF_PAD), 1) & (RBF_PAD - 1)
          ).astype(f32) * step
    d4 = d_ref[...]
    bsel = (lax.broadcasted_iota(i32, (_EPR, _EPR * RBF_PAD), 1) // RBF_PAD
            == lax.broadcasted_iota(i32, (_EPR, _EPR * RBF_PAD), 0)).astype(f32)
    dm = jnp.dot(d4, bsel, preferred_element_type=f32,
                 precision=lax.Precision.HIGHEST)
    env = jnp.where(dm <= CUTOFF,
                    0.5 * (jnp.cos(np.pi * dm / CUTOFF) + 1.0), 0.0)
    ef4 = jnp.exp(-((dm - mu) ** 2) * inv2s2) * env
    dd4 = jnp.dot(ef4, wd_ref[...], preferred_element_type=f32) + bd_ref[...]
    dd4 = _swish(dd4)
    dd = jnp.concatenate(
        [jnp.concatenate(
            [dd4[:, (g * _EPR + s) * FEAT:(g * _EPR + s + 1) * FEAT]
             for g in range(4)], axis=1)
         for s in range(_EPR)], axis=0)
    qw = qg_ref[...]
    kvw = kvg_ref[...]
    kw = kvw[:, :FEAT]
    vw = kvw[:, FEAT:]
    prod0 = _unpack_lo(qw) * _unpack_lo(kw) * dd[:, :FEAT]
    prod1 = _unpack_hi(qw) * _unpack_hi(kw) * dd[:, FEAT:2 * FEAT]
    w0 = _swish(jnp.sum(prod0, axis=1, keepdims=True))
    w1 = _swish(jnp.sum(prod1, axis=1, keepdims=True))
    out_ref[:, :FEAT] = _unpack_lo(vw) * dd[:, 2 * FEAT:3 * FEAT] * w0
    out_ref[:, FEAT:] = _unpack_hi(vw) * dd[:, 3 * FEAT:] * w1


def _edge_stage(out_prev, blk0, e_out, dist4, qg, kvg, wd_p, bd_p):
    esl = qg.shape[0]
    be = _EPR * _BR
    nblk_total = (e_out + be - 1) // be
    grid = min(esl // be, nblk_total - blk0)
    if grid <= 0:
        return out_prev
    sigma = CUTOFF / (N_RBF - 1)
    body = functools.partial(_edge_body, float(sigma),
                             float(1.0 / (2.0 * sigma * sigma)))
    in_specs = [
        pl.BlockSpec((_BR, _EPR), lambda i: (i + blk0, 0)),
        pl.BlockSpec((be, FEAT), lambda i: (i, 0)),
        pl.BlockSpec((be, 2 * FEAT), lambda i: (i, 0)),
        pl.BlockSpec((_EPR * RBF_PAD, 4 * _EPR * FEAT), lambda i: (0, 0)),
        pl.BlockSpec((1, 4 * _EPR * FEAT), lambda i: (0, 0)),
    ]
    args = (dist4, qg, kvg, wd_p, bd_p)
    aliases = {}
    if out_prev is not None:
        in_specs = [pl.BlockSpec(memory_space=pl.ANY)] + in_specs
        args = (out_prev,) + args
        aliases = {0: 0}
    return pl.pallas_call(
        body,
        grid=(grid,),
        in_specs=in_specs,
        out_specs=pl.BlockSpec((be, HEADS * FEAT), lambda i: (i + blk0, 0)),
        out_shape=jax.ShapeDtypeStruct((e_out, HEADS * FEAT), jnp.float32),
        input_output_aliases=aliases,
    )(*args)


def _pack_node_w(w):
    return jnp.transpose(w, (2, 0, 1)).reshape(w.shape[2], w.shape[0] * w.shape[1])


def kernel(dist, nbrs, x_i, Wq, bq, Wk, bk, Wdk, bdk, Wv, bv, Wdv, bdv):
    e = dist.shape[0]
    n = x_i.shape[0]
    unit = _NW * _CHUNK * _NBUF
    e_pad = ((e + unit - 1) // unit) * unit
    units = e_pad // unit
    rest_units = units - 1
    n_rest = max(1, rest_units // 4)
    base_u, extra = divmod(rest_units, n_rest) if rest_units else (0, 0)
    sizes = [unit] + [(base_u + (i < extra)) * unit for i in range(n_rest)]
    sizes = [sz for sz in sizes if sz > 0]

    wq_p = _pack_node_w(Wq)
    wkv_p = jnp.concatenate([_pack_node_w(Wk), _pack_node_w(Wv)], axis=1)
    bq_p = bq.reshape(1, HEADS * FEAT)
    bkv_p = jnp.concatenate([bk.reshape(1, -1), bv.reshape(1, -1)], axis=1)

    wdk_p = _pack_node_w(Wdk)
    wdv_p = _pack_node_w(Wdv)
    wd_p = jnp.zeros((RBF_PAD, 2 * HEADS * FEAT), jnp.float32)
    wd_p = wd_p.at[:N_RBF].set(jnp.concatenate([wdk_p, wdv_p], axis=1))
    bd_p = jnp.concatenate([bdk.reshape(1, -1), bdv.reshape(1, -1)], axis=1)
    eye4 = jnp.eye(_EPR, dtype=jnp.float32)
    wd_r = wd_p.reshape(RBF_PAD, 4, FEAT)
    wblk = (eye4[:, None, None, :, None] * wd_r[None, :, :, None, :]).reshape(
        _EPR * RBF_PAD, 4 * _EPR * FEAT)
    bd4 = jnp.broadcast_to(bd_p.reshape(4, 1, FEAT), (4, _EPR, FEAT)).reshape(
        1, 4 * _EPR * FEAT)

    qtab_w, kvtab_w = _node_proj(x_i, wq_p, bq_p, wkv_p, bkv_p)

    idx = nbrs.astype(jnp.int32)
    pad = e_pad - e
    padvals = (jnp.arange(pad, dtype=jnp.int32) * 37) % n
    idx_i = jnp.concatenate([idx[:, 0], padvals])
    idx_j = jnp.concatenate([idx[:, 1], padvals])
    dist4 = jnp.pad(dist, (0, pad), constant_values=1.0).reshape(
        e_pad // (_EPR * _BR), _EPR, _BR).transpose(0, 2, 1).reshape(
        e_pad // _EPR, _EPR)

    gath_by_size = {}
    gathered = []
    lo = 0
    for sz in sizes:
        if sz not in gath_by_size:
            gath_by_size[sz] = _make_gather(sz)
        gathered.append(gath_by_size[sz](
            qtab_w, kvtab_w,
            lax.slice(idx_i, (lo,), (lo + sz,)),
            lax.slice(idx_j, (lo,), (lo + sz,))))
        lo += sz
    out = None
    blk0 = 0
    be = _EPR * _BR
    for sz, (qg_w, kvg_w) in zip(sizes, gathered):
        out = _edge_stage(out, blk0, e, dist4, qg_w, kvg_w, wblk, bd4)
        blk0 += sz // be
    return out


# speedup vs baseline: 1.0519x; 1.0138x over previous
import functools

import jax
import jax.numpy as jnp
import numpy as np
from jax import lax
from jax.experimental import pallas as pl
from jax.experimental.pallas import tpu as pltpu
from jax.experimental.pallas import tpu_sc as plsc

FEAT = 128
HEADS = 2
N_RBF = 20
CUTOFF = 5.0
RBF_PAD = 32

_NC = 2
_NS = 16
_NW = _NC * _NS
_CHUNK = 128
_NBUF = 2


_LOG2E = 1.4426950408889634


def _swish(x):
    return x / (1.0 + jnp.exp2(x * -_LOG2E))


def _bf16_bits(x):
    b = lax.bitcast_convert_type(x, jnp.int32)
    return lax.shift_right_logical(
        b + 0x7FFF + (lax.shift_right_logical(b, 16) & 1), 16)


def _pack_pair(lo, hi):
    return _bf16_bits(lo) | lax.shift_left(_bf16_bits(hi), 16)


def _unpack_lo(w):
    return lax.bitcast_convert_type(lax.shift_left(w, 16), jnp.float32)


def _unpack_hi(w):
    return lax.bitcast_convert_type(w & jnp.int32(-65536), jnp.float32)


def _proj_body(x_ref, wq_ref, bq_ref, wkv_ref, bkv_ref, q_ref, kv_ref):
    x = x_ref[...]
    q = jnp.dot(x, wq_ref[...], preferred_element_type=jnp.float32) + bq_ref[...]
    kv = jnp.dot(x, wkv_ref[...], preferred_element_type=jnp.float32) + bkv_ref[...]
    q_ref[...] = _pack_pair(q[:, :FEAT], q[:, FEAT:])
    kv_ref[:, :FEAT] = _pack_pair(kv[:, :FEAT], kv[:, FEAT:2 * FEAT])
    kv_ref[:, FEAT:] = _pack_pair(kv[:, 2 * FEAT:3 * FEAT], kv[:, 3 * FEAT:])


def _node_proj(x_i, wq_p, bq_p, wkv_p, bkv_p):
    n = x_i.shape[0]
    blk = 1000
    grid = n // blk
    return pl.pallas_call(
        _proj_body,
        grid=(grid,),
        in_specs=[
            pl.BlockSpec((blk, FEAT), lambda i: (i, 0)),
            pl.BlockSpec((FEAT, HEADS * FEAT), lambda i: (0, 0)),
            pl.BlockSpec((1, HEADS * FEAT), lambda i: (0, 0)),
            pl.BlockSpec((FEAT, 2 * HEADS * FEAT), lambda i: (0, 0)),
            pl.BlockSpec((1, 2 * HEADS * FEAT), lambda i: (0, 0)),
        ],
        out_specs=[
            pl.BlockSpec((blk, FEAT), lambda i: (i, 0)),
            pl.BlockSpec((blk, 2 * FEAT), lambda i: (i, 0)),
        ],
        out_shape=[
            jax.ShapeDtypeStruct((n, FEAT), jnp.int32),
            jax.ShapeDtypeStruct((n, 2 * FEAT), jnp.int32),
        ],
    )(x_i, wq_p, bq_p, wkv_p, bkv_p)


def _make_gather(e_pad):
    epw = e_pad // _NW
    nch = epw // _CHUNK
    nouter = nch // _NBUF
    mesh = plsc.VectorSubcoreMesh(
        core_axis_name="c", subcore_axis_name="s",
        num_cores=_NC, num_subcores=_NS)

    dq = HEADS * FEAT // 2
    dkv = HEADS * FEAT

    @functools.partial(
        pl.kernel,
        mesh=mesh,
        out_type=[
            jax.ShapeDtypeStruct((e_pad, dq), jnp.int32),
            jax.ShapeDtypeStruct((e_pad, dkv), jnp.int32),
        ],
        scratch_types=[
            pltpu.VMEM((epw,), jnp.int32),
            pltpu.VMEM((epw,), jnp.int32),
            pltpu.VMEM((_NBUF, _CHUNK, dq), jnp.int32),
            pltpu.VMEM((_NBUF, _CHUNK, dkv), jnp.int32),
        ] + [pltpu.SemaphoreType.DMA] * (4 * _NBUF),
    )
    def gather_k(qtab, kvtab, idx_i, idx_j, qout, kvout,
                 idxi_v, idxj_v, qbuf, kvbuf, *sems):
        sq = sems[0:_NBUF]
        skv = sems[_NBUF:2 * _NBUF]
        wq = sems[2 * _NBUF:3 * _NBUF]
        wkv = sems[3 * _NBUF:4 * _NBUF]
        wid = lax.axis_index("s") * _NC + lax.axis_index("c")
        base = pl.multiple_of(wid * epw, _CHUNK)
        pltpu.sync_copy(idx_i.at[pl.ds(base, epw)], idxi_v)
        pltpu.sync_copy(idx_j.at[pl.ds(base, epw)], idxj_v)

        def issue(b, c):
            pltpu.async_copy(
                qtab.at[idxi_v.at[pl.ds(c * _CHUNK, _CHUNK)]], qbuf.at[b], sq[b])
            pltpu.async_copy(
                kvtab.at[idxj_v.at[pl.ds(c * _CHUNK, _CHUNK)]], kvbuf.at[b], skv[b])

        def wait_gather(b):
            pltpu.make_async_copy(
                qtab.at[idxi_v.at[pl.ds(0, _CHUNK)]], qbuf.at[b], sq[b]).wait()
            pltpu.make_async_copy(
                kvtab.at[idxj_v.at[pl.ds(0, _CHUNK)]], kvbuf.at[b], skv[b]).wait()

        def write(b, c):
            off = pl.multiple_of(base + c * _CHUNK, _CHUNK)
            pltpu.async_copy(qbuf.at[b], qout.at[pl.ds(off, _CHUNK)], wq[b])
            pltpu.async_copy(kvbuf.at[b], kvout.at[pl.ds(off, _CHUNK)], wkv[b])

        def wait_write(b):
            pltpu.make_async_copy(
                qbuf.at[b], qout.at[pl.ds(0, _CHUNK)], wq[b]).wait()
            pltpu.make_async_copy(
                kvbuf.at[b], kvout.at[pl.ds(0, _CHUNK)], wkv[b]).wait()

        for b in range(_NBUF):
            issue(b, b)

        def outer(g, carry):
            for b in range(_NBUF):
                wait_gather(b)
                write(b, g * _NBUF + b)
            for b in range(_NBUF):
                wait_write(b)
                nxt = (g + 1) * _NBUF + b

                @pl.when(nxt < nch)
                def _():
                    issue(b, nxt)
            return carry

        lax.fori_loop(0, nouter, outer, 0)

    return gather_k


_EPR = 4
_BR = 512


def _edge_body(step, inv2s2, *refs):
    if len(refs) == 7:
        refs = refs[1:]
    d_ref, qg_ref, kvg_ref, wd_ref, bd_ref, out_ref = refs
    f32, i32 = jnp.float32, jnp.int32
    be = out_ref.shape[0]
    br = be // _EPR
    mu = (lax.broadcasted_iota(i32, (1, _EPR * RBF_PAD), 1) & (RBF_PAD - 1)
          ).astype(f32) * step
    d4 = d_ref[...]
    bsel = (lax.broadcasted_iota(i32, (_EPR, _EPR * RBF_PAD), 1) // RBF_PAD
            == lax.broadcasted_iota(i32, (_EPR, _EPR * RBF_PAD), 0)).astype(f32)
    dm = jnp.dot(d4, bsel, preferred_element_type=f32,
                 precision=lax.Precision.HIGHEST)
    env = jnp.where(dm <= CUTOFF,
                    0.5 * (jnp.cos(np.pi * dm / CUTOFF) + 1.0), 0.0)
    ef4 = jnp.exp(-((dm - mu) ** 2) * inv2s2) * env
    dd4 = jnp.dot(ef4, wd_ref[...], preferred_element_type=f32) + bd_ref[...]
    dd4 = _swish(dd4)
    dd = jnp.concatenate(
        [jnp.concatenate(
            [dd4[:, (g * _EPR + s) * FEAT:(g * _EPR + s + 1) * FEAT]
             for g in range(4)], axis=1)
         for s in range(_EPR)], axis=0)
    qw = qg_ref[...]
    kvw = kvg_ref[...]
    kw = kvw[:, :FEAT]
    vw = kvw[:, FEAT:]
    prod0 = _unpack_lo(qw) * _unpack_lo(kw) * dd[:, :FEAT]
    prod1 = _unpack_hi(qw) * _unpack_hi(kw) * dd[:, FEAT:2 * FEAT]
    w0 = _swish(jnp.sum(prod0, axis=1, keepdims=True))
    w1 = _swish(jnp.sum(prod1, axis=1, keepdims=True))
    out_ref[:, :FEAT] = _unpack_lo(vw) * dd[:, 2 * FEAT:3 * FEAT] * w0
    out_ref[:, FEAT:] = _unpack_hi(vw) * dd[:, 3 * FEAT:] * w1


def _edge_stage(out_prev, blk0, e_out, dist4, qg, kvg, wd_p, bd_p):
    esl = qg.shape[0]
    be = _EPR * _BR
    nblk_total = (e_out + be - 1) // be
    grid = min(esl // be, nblk_total - blk0)
    if grid <= 0:
        return out_prev
    sigma = CUTOFF / (N_RBF - 1)
    body = functools.partial(_edge_body, float(sigma),
                             float(1.0 / (2.0 * sigma * sigma)))
    in_specs = [
        pl.BlockSpec((_BR, _EPR), lambda i: (i + blk0, 0)),
        pl.BlockSpec((be, FEAT), lambda i: (i, 0)),
        pl.BlockSpec((be, 2 * FEAT), lambda i: (i, 0)),
        pl.BlockSpec((_EPR * RBF_PAD, 4 * _EPR * FEAT), lambda i: (0, 0)),
        pl.BlockSpec((1, 4 * _EPR * FEAT), lambda i: (0, 0)),
    ]
    args = (dist4, qg, kvg, wd_p, bd_p)
    aliases = {}
    if out_prev is not None:
        in_specs = [pl.BlockSpec(memory_space=pl.ANY)] + in_specs
        args = (out_prev,) + args
        aliases = {0: 0}
    return pl.pallas_call(
        body,
        grid=(grid,),
        in_specs=in_specs,
        out_specs=pl.BlockSpec((be, HEADS * FEAT), lambda i: (i + blk0, 0)),
        out_shape=jax.ShapeDtypeStruct((e_out, HEADS * FEAT), jnp.float32),
        input_output_aliases=aliases,
    )(*args)


def _pack_node_w(w):
    return jnp.transpose(w, (2, 0, 1)).reshape(w.shape[2], w.shape[0] * w.shape[1])


def kernel(dist, nbrs, x_i, Wq, bq, Wk, bk, Wdk, bdk, Wv, bv, Wdv, bdv):
    e = dist.shape[0]
    n = x_i.shape[0]
    n_sl = 5
    unit = _NW * _CHUNK * _NBUF
    esl = ((e + n_sl * unit - 1) // (n_sl * unit)) * unit
    e_pad = n_sl * esl
    sizes = [esl] * n_sl

    wq_p = _pack_node_w(Wq)
    wkv_p = jnp.concatenate([_pack_node_w(Wk), _pack_node_w(Wv)], axis=1)
    bq_p = bq.reshape(1, HEADS * FEAT)
    bkv_p = jnp.concatenate([bk.reshape(1, -1), bv.reshape(1, -1)], axis=1)

    wdk_p = _pack_node_w(Wdk)
    wdv_p = _pack_node_w(Wdv)
    wd_p = jnp.zeros((RBF_PAD, 2 * HEADS * FEAT), jnp.float32)
    wd_p = wd_p.at[:N_RBF].set(jnp.concatenate([wdk_p, wdv_p], axis=1))
    bd_p = jnp.concatenate([bdk.reshape(1, -1), bdv.reshape(1, -1)], axis=1)
    eye4 = jnp.eye(_EPR, dtype=jnp.float32)
    wd_r = wd_p.reshape(RBF_PAD, 4, FEAT)
    wblk = (eye4[:, None, None, :, None] * wd_r[None, :, :, None, :]).reshape(
        _EPR * RBF_PAD, 4 * _EPR * FEAT)
    bd4 = jnp.broadcast_to(bd_p.reshape(4, 1, FEAT), (4, _EPR, FEAT)).reshape(
        1, 4 * _EPR * FEAT)

    qtab_w, kvtab_w = _node_proj(x_i, wq_p, bq_p, wkv_p, bkv_p)

    idx = nbrs.astype(jnp.int32)
    pad = e_pad - e
    padvals = (jnp.arange(pad, dtype=jnp.int32) * 37) % n
    idx_i = jnp.concatenate([idx[:, 0], padvals])
    idx_j = jnp.concatenate([idx[:, 1], padvals])
    dist4 = jnp.pad(dist, (0, pad), constant_values=1.0).reshape(
        e_pad // (_EPR * _BR), _EPR, _BR).transpose(0, 2, 1).reshape(
        e_pad // _EPR, _EPR)

    gath_by_size = {}
    gathered = []
    lo = 0
    for sz in sizes:
        if sz not in gath_by_size:
            gath_by_size[sz] = _make_gather(sz)
        gathered.append(gath_by_size[sz](
            qtab_w, kvtab_w,
            lax.slice(idx_i, (lo,), (lo + sz,)),
            lax.slice(idx_j, (lo,), (lo + sz,))))
        lo += sz
    out = None
    blk0 = 0
    be = _EPR * _BR
    for sz, (qg_w, kvg_w) in zip(sizes, gathered):
        out = _edge_stage(out, blk0, e, dist4, qg_w, kvg_w, wblk, bd4)
        blk0 += sz // be
    return out
